# Initial kernel scaffold; baseline (speedup 1.0000x reference)
import functools

import jax
import jax.numpy as jnp
from jax import lax
from jax.experimental import pallas as pl
from jax.experimental.pallas import tpu as pltpu
from jax.experimental.pallas import tpu_sc as plsc

C = 1.0
MIN_NORM = 1e-15
EPS = 4e-3

NUM_CORES = 2
NUM_SUBCORES = 16
NW = NUM_CORES * NUM_SUBCORES
LANES = 16

K_EDGES = 80


def _artanh(x):
  return jnp.arctanh(jnp.clip(x, -1.0 + 1e-7, 1.0 - 1e-7))


def _norm(x):
  return jnp.clip(
      jnp.sqrt(jnp.sum(x * x, axis=-1, keepdims=True)), MIN_NORM, None
  )


def _proj(x):
  norm = _norm(x)
  maxnorm = (1.0 - EPS) / (C**0.5)
  return jnp.where(norm > maxnorm, x / norm * maxnorm, x)


def _dense_body(x_ref, w_ref, b_ref, out_ref):
  x = x_ref[...]
  w = w_ref[...]
  bias = b_ref[...]

  x_norm = _norm(x)
  mx = lax.dot_general(
      x,
      w,
      (((1,), (1,)), ((), ())),
      preferred_element_type=jnp.float32,
      precision=lax.Precision.HIGHEST,
  )
  mx_norm = _norm(mx)
  res_c = jnp.tanh(mx_norm / x_norm * _artanh(x_norm)) * mx / mx_norm
  cond = jnp.all(mx == 0, axis=-1, keepdims=True)
  h = jnp.where(cond, jnp.zeros_like(res_c), res_c)
  h = _proj(h)

  b_norm = _norm(bias)
  hyp_b = jnp.tanh(b_norm) * bias / b_norm

  x2 = jnp.sum(h * h, axis=-1, keepdims=True)
  y2 = jnp.sum(hyp_b * hyp_b, axis=-1, keepdims=True)
  xy = jnp.sum(h * hyp_b, axis=-1, keepdims=True)
  num = (1.0 + 2.0 * xy + y2) * h + (1.0 - x2) * hyp_b
  denom = 1.0 + 2.0 * xy + x2 * y2
  h = num / jnp.clip(denom, MIN_NORM, None)
  h = _proj(h)

  p_norm = _norm(h)
  out_ref[...] = (_artanh(p_norm) / p_norm) * h


def _dense_stage(x, weight, bias, blk):
  n, d = x.shape
  grid = n // blk
  return pl.pallas_call(
      _dense_body,
      grid=(grid,),
      in_specs=[
          pl.BlockSpec((blk, d), lambda i: (i, 0)),
          pl.BlockSpec((d, d), lambda i: (0, 0)),
          pl.BlockSpec((1, d), lambda i: (0, 0)),
      ],
      out_specs=pl.BlockSpec((blk, d), lambda i: (i, 0)),
      out_shape=jax.ShapeDtypeStruct((n, d), jnp.float32),
  )(x, weight, bias.reshape(1, d))


def _combine_body(p_ref, out_ref):
  out_ref[...] = jnp.maximum(p_ref[0] + p_ref[1], 0.0)


def _combine_stage(partials, blk):
  _, n, d = partials.shape
  return pl.pallas_call(
      _combine_body,
      grid=(n // blk,),
      in_specs=[pl.BlockSpec((2, blk, d), lambda i: (0, i, 0))],
      out_specs=pl.BlockSpec((blk, d), lambda i: (i, 0)),
      out_shape=jax.ShapeDtypeStruct((n, d), jnp.float32),
  )(partials)


def _sc_body(n, d, edges_per_tile, src_hbm, dst_hbm, adj_hbm, ht_hbm,
             zeros_hbm, out_hbm, src_v, dst_v, adj_v, rows_v, acc_sh, sem):
  cid = lax.axis_index("c")
  sid = lax.axis_index("s")
  wid = cid * NUM_SUBCORES + sid

  rows_per_tile = n // NUM_SUBCORES
  r0 = sid * rows_per_tile
  pltpu.sync_copy(
      zeros_hbm.at[pl.ds(r0, rows_per_tile)],
      acc_sh.at[pl.ds(r0, rows_per_tile)],
  )
  plsc.subcore_barrier()

  num_batches = edges_per_tile // K_EDGES
  edge0 = wid * edges_per_tile

  def batch(b, carry):
    base = edge0 + b * K_EDGES
    pltpu.sync_copy(src_hbm.at[pl.ds(base, K_EDGES)], src_v)
    pltpu.sync_copy(dst_hbm.at[pl.ds(base, K_EDGES)], dst_v)
    pltpu.sync_copy(adj_hbm.at[pl.ds(base, K_EDGES)], adj_v)
    pltpu.async_copy(ht_hbm.at[src_v], rows_v, sem).wait()
    for i in range(K_EDGES):
      a = plsc.load_gather(adj_v, [jnp.full((LANES,), i, jnp.int32)])
      for j in range(d // LANES):
        sl = pl.ds(j * LANES, LANES)
        rows_v[i, sl] = rows_v[i, sl] * a
    pltpu.sync_copy(rows_v, acc_sh.at[dst_v], add=True)
    return carry

  lax.fori_loop(0, num_batches, batch, 0)

  plsc.subcore_barrier()
  pltpu.sync_copy(
      acc_sh.at[pl.ds(r0, rows_per_tile)],
      out_hbm.at[cid, pl.ds(r0, rows_per_tile)],
  )


def _sc_stage(src, dst, adj, h_t, zeros, n, d, edges_per_tile):
  mesh = plsc.VectorSubcoreMesh(
      core_axis_name="c", subcore_axis_name="s"
  )
  body = functools.partial(_sc_body, n, d, edges_per_tile)
  return pl.kernel(
      body,
      out_type=jax.ShapeDtypeStruct((NUM_CORES, n, d), jnp.float32),
      mesh=mesh,
      scratch_types=[
          pltpu.VMEM((K_EDGES,), jnp.int32),
          pltpu.VMEM((K_EDGES,), jnp.int32),
          pltpu.VMEM((K_EDGES,), jnp.float32),
          pltpu.VMEM((K_EDGES, d), jnp.float32),
          pltpu.VMEM_SHARED((n, d), jnp.float32),
          pltpu.SemaphoreType.DMA,
      ],
  )(src, dst, adj, h_t, zeros)


def kernel(x, edge_index, adj_vals, weight, bias):
  n, d = x.shape
  e = edge_index.shape[1]

  h_t = _dense_stage(x, weight, bias, blk=1000)

  src = edge_index[0].astype(jnp.int32)
  dst = edge_index[1].astype(jnp.int32)
  adj = adj_vals.astype(jnp.float32)

  chunk = NW * K_EDGES
  e_pad = ((e + chunk - 1) // chunk) * chunk
  if e_pad != e:
    pad = e_pad - e
    src = jnp.concatenate([src, jnp.zeros((pad,), jnp.int32)])
    dst = jnp.concatenate([dst, jnp.zeros((pad,), jnp.int32)])
    adj = jnp.concatenate([adj, jnp.zeros((pad,), jnp.float32)])

  zeros = jnp.zeros((n, d), jnp.float32)
  partials = _sc_stage(src, dst, adj, h_t, zeros, n, d, e_pad // NW)

  return _combine_stage(partials, blk=1000)


# speedup vs baseline: 4.0605x; 4.0605x over previous
import functools

import jax
import jax.numpy as jnp
from jax import lax
from jax.experimental import pallas as pl
from jax.experimental.pallas import tpu as pltpu
from jax.experimental.pallas import tpu_sc as plsc

C = 1.0
MIN_NORM = 1e-15
EPS = 4e-3

NUM_CORES = 2
NUM_SUBCORES = 16
NW = NUM_CORES * NUM_SUBCORES
LANES = 16

K_EDGES = 96

_BCAST_DNUMS = lax.GatherDimensionNumbers(
    offset_dims=(), collapsed_slice_dims=(0,), start_index_map=(0,))


def _lane_bcast(vec, lane):
  idx = jnp.full((LANES, 1), lane, jnp.int32)
  return lax.gather(vec, idx, _BCAST_DNUMS, (1,),
                    mode=lax.GatherScatterMode.PROMISE_IN_BOUNDS)


def _artanh(x):
  x = jnp.clip(x, -1.0 + 1e-7, 1.0 - 1e-7)
  return 0.5 * (jnp.log1p(x) - jnp.log1p(-x))


def _norm(x):
  return jnp.clip(
      jnp.sqrt(jnp.sum(x * x, axis=-1, keepdims=True)), MIN_NORM, None
  )


def _proj(x):
  norm = _norm(x)
  maxnorm = (1.0 - EPS) / (C**0.5)
  return jnp.where(norm > maxnorm, x / norm * maxnorm, x)


def _dense_body(x_ref, w_ref, b_ref, out_ref):
  x = x_ref[...]
  w = w_ref[...]
  bias = b_ref[...]

  x_norm = _norm(x)
  mx = lax.dot_general(
      x,
      w,
      (((1,), (1,)), ((), ())),
      preferred_element_type=jnp.float32,
      precision=lax.Precision.HIGHEST,
  )
  mx_norm = _norm(mx)
  res_c = jnp.tanh(mx_norm / x_norm * _artanh(x_norm)) * mx / mx_norm
  cond = jnp.all(mx == 0, axis=-1, keepdims=True)
  h = jnp.where(cond, jnp.zeros_like(res_c), res_c)
  h = _proj(h)

  b_norm = _norm(bias)
  hyp_b = jnp.tanh(b_norm) * bias / b_norm

  x2 = jnp.sum(h * h, axis=-1, keepdims=True)
  y2 = jnp.sum(hyp_b * hyp_b, axis=-1, keepdims=True)
  xy = jnp.sum(h * hyp_b, axis=-1, keepdims=True)
  num = (1.0 + 2.0 * xy + y2) * h + (1.0 - x2) * hyp_b
  denom = 1.0 + 2.0 * xy + x2 * y2
  h = num / jnp.clip(denom, MIN_NORM, None)
  h = _proj(h)

  p_norm = _norm(h)
  out_ref[...] = (_artanh(p_norm) / p_norm) * h


def _dense_stage(x, weight, bias, blk):
  n, d = x.shape
  grid = n // blk
  return pl.pallas_call(
      _dense_body,
      grid=(grid,),
      in_specs=[
          pl.BlockSpec((blk, d), lambda i: (i, 0)),
          pl.BlockSpec((d, d), lambda i: (0, 0)),
          pl.BlockSpec((1, d), lambda i: (0, 0)),
      ],
      out_specs=pl.BlockSpec((blk, d), lambda i: (i, 0)),
      out_shape=jax.ShapeDtypeStruct((n, d), jnp.float32),
  )(x, weight, bias.reshape(1, d))


def _combine_body(p_ref, out_ref):
  out_ref[...] = jnp.maximum(p_ref[0] + p_ref[1], 0.0)


def _combine_stage(partials, blk):
  _, n, d = partials.shape
  return pl.pallas_call(
      _combine_body,
      grid=(n // blk,),
      in_specs=[pl.BlockSpec((2, blk, d), lambda i: (0, i, 0))],
      out_specs=pl.BlockSpec((blk, d), lambda i: (i, 0)),
      out_shape=jax.ShapeDtypeStruct((n, d), jnp.float32),
  )(partials)


def _sc_body(n, d, num_batches, ed_hbm, ht_hbm, zeros_hbm, out_hbm,
             ed_v, rows_v, acc_sh, gsem):
  cid = lax.axis_index("c")
  sid = lax.axis_index("s")
  wid = cid * NUM_SUBCORES + sid

  rows_per_tile = n // NUM_SUBCORES
  r0 = sid * rows_per_tile
  pltpu.sync_copy(
      zeros_hbm.at[pl.ds(r0, rows_per_tile)],
      acc_sh.at[pl.ds(r0, rows_per_tile)],
  )
  plsc.subcore_barrier()

  b0 = wid * num_batches

  def start_gather(slot):
    pltpu.async_copy(ht_hbm.at[ed_v.at[slot, 0]], rows_v.at[slot], gsem)

  def wait_gather(slot):
    pltpu.make_async_copy(
        ht_hbm.at[ed_v.at[slot, 0]], rows_v.at[slot], gsem
    ).wait()

  pltpu.sync_copy(ed_hbm.at[b0], ed_v.at[0])
  start_gather(0)
  if num_batches > 1:
    pltpu.sync_copy(ed_hbm.at[b0 + 1], ed_v.at[1])

  def process(b, slot):
    wait_gather(slot)

    @pl.when(b + 1 < num_batches)
    def _():
      start_gather(1 - slot)

    for g in range(K_EDGES // LANES):
      av = plsc.bitcast(ed_v[slot, 2, pl.ds(g * LANES, LANES)], jnp.float32)
      for l in range(LANES):
        a = _lane_bcast(av, l)
        row = g * LANES + l
        for j in range(d // LANES):
          sl = pl.ds(j * LANES, LANES)
          rows_v[slot, row, sl] = rows_v[slot, row, sl] * a

    pltpu.sync_copy(rows_v.at[slot], acc_sh.at[ed_v.at[slot, 1]], add=True)

    @pl.when(b + 2 < num_batches)
    def _():
      pltpu.sync_copy(ed_hbm.at[b0 + b + 2], ed_v.at[slot])

  def batch_pair(bb, carry):
    process(bb * 2, 0)
    process(bb * 2 + 1, 1)
    return carry

  lax.fori_loop(0, num_batches // 2, batch_pair, 0)

  plsc.subcore_barrier()
  pltpu.sync_copy(
      acc_sh.at[pl.ds(r0, rows_per_tile)],
      out_hbm.at[cid, pl.ds(r0, rows_per_tile)],
  )


def _sc_stage(ed, h_t, zeros, n, d, num_batches):
  mesh = plsc.VectorSubcoreMesh(
      core_axis_name="c", subcore_axis_name="s"
  )
  body = functools.partial(_sc_body, n, d, num_batches)
  return pl.kernel(
      body,
      out_type=jax.ShapeDtypeStruct((NUM_CORES, n, d), jnp.float32),
      mesh=mesh,
      compiler_params=pltpu.CompilerParams(needs_layout_passes=False),
      scratch_types=[
          pltpu.VMEM((2, 4, K_EDGES), jnp.int32),
          pltpu.VMEM((2, K_EDGES, d), jnp.float32),
          pltpu.VMEM_SHARED((n, d), jnp.float32),
          pltpu.SemaphoreType.DMA,
      ],
  )(ed, h_t, zeros)


def kernel(x, edge_index, adj_vals, weight, bias):
  n, d = x.shape
  e = edge_index.shape[1]

  h_t = _dense_stage(x, weight, bias, blk=1000)

  src = edge_index[0].astype(jnp.int32)
  dst = edge_index[1].astype(jnp.int32)
  adj = adj_vals.astype(jnp.float32)

  chunk = NW * K_EDGES * 2
  e_pad = ((e + chunk - 1) // chunk) * chunk
  if e_pad != e:
    pad = e_pad - e
    src = jnp.concatenate([src, jnp.zeros((pad,), jnp.int32)])
    dst = jnp.concatenate([dst, jnp.zeros((pad,), jnp.int32)])
    adj = jnp.concatenate([adj, jnp.zeros((pad,), jnp.float32)])
  nb = e_pad // (NW * K_EDGES)

  adj_bits = lax.bitcast_convert_type(adj, jnp.int32)
  ed = jnp.stack([src, dst, adj_bits])
  ed = ed.reshape(3, NW, nb, K_EDGES).transpose(1, 2, 0, 3)
  ed = jnp.concatenate(
      [ed, jnp.zeros((NW, nb, 1, K_EDGES), jnp.int32)], axis=2)
  ed = ed.reshape(NW * nb, 4, K_EDGES)

  row_chunk = NUM_SUBCORES * 8
  n_pad = ((n + row_chunk - 1) // row_chunk) * row_chunk

  zeros = jnp.zeros((n_pad, d), jnp.float32)
  partials = _sc_stage(ed, h_t, zeros, n_pad, d, nb)

  out = _combine_stage(partials, blk=n_pad // NUM_SUBCORES)
  return out[:n]
